# baseline (speedup 1.0000x reference)
import functools
import jax
import jax.numpy as jnp
from jax import lax
from jax.experimental import pallas as pl
from jax.experimental.pallas import tpu as pltpu
from jax.experimental.pallas import tpu_sc as plsc


def kernel(batch_mask, mask_emb):
    M, N = batch_mask.shape
    _, D = mask_emb.shape
    B = M * N
    NC, NS, L = 2, 16, 16
    T = B * D
    half = T // NC
    SPB = 1024 * 1024
    n_dma = half // SPB

    idx = batch_mask.reshape(B)

    mesh = plsc.VectorSubcoreMesh(
        core_axis_name="c", subcore_axis_name="s", num_cores=NC, num_subcores=NS
    )

    @functools.partial(
        pl.kernel,
        mesh=mesh,
        out_type=jax.ShapeDtypeStruct((T,), jnp.float32),
        scratch_types=[
            pltpu.VMEM_SHARED((SPB,), jnp.float32),
            pltpu.SemaphoreType.DMA,
        ],
    )
    def k(idx_hbm, out_hbm, sh_v, so):
        cid = lax.axis_index("c")
        sid = lax.axis_index("s")
        base = cid * half

        @pl.when(sid == 0)
        def _driver():
            def step(i, carry):
                pltpu.async_copy(
                    sh_v, out_hbm.at[pl.ds(base + i * SPB, SPB)], so
                )

                @pl.when(i >= 2)
                def _drain():
                    pltpu.make_async_copy(
                        sh_v, out_hbm.at[pl.ds(base, SPB)], so
                    ).wait()

                return carry

            lax.fori_loop(0, n_dma, step, 0)
            for _ in range(2):
                pltpu.make_async_copy(
                    sh_v, out_hbm.at[pl.ds(base, SPB)], so
                ).wait()

    out = k(idx)
    return out.reshape(M, N, D)


# speedup vs baseline: 2.1002x; 2.1002x over previous
import jax
import jax.numpy as jnp
from jax.experimental import pallas as pl


def _body(m_ref, k_ref, ee_ref, out_ref):
    m = m_ref[...].astype(jnp.float32)
    y = jnp.dot(m, k_ref[...], preferred_element_type=jnp.float32)
    out_ref[...] = y + ee_ref[0, :][None, :]


def tc_kernel(batch_mask, mask_emb, bm=256):
    M, N = batch_mask.shape
    _, D = mask_emb.shape
    W = N * D
    diff = mask_emb[1] - mask_emb[0]
    K = (jnp.eye(N, dtype=jnp.float32)[:, :, None] * diff[None, None, :]).reshape(N, W)
    ee = jnp.tile(mask_emb[0], N)[None, :]

    out = pl.pallas_call(
        _body,
        grid=(M // bm,),
        in_specs=[
            pl.BlockSpec((bm, N), lambda i: (i, 0)),
            pl.BlockSpec((N, W), lambda i: (0, 0)),
            pl.BlockSpec((1, W), lambda i: (0, 0)),
        ],
        out_specs=pl.BlockSpec((bm, W), lambda i: (i, 0)),
        out_shape=jax.ShapeDtypeStruct((M, W), jnp.float32),
    )(batch_mask, K, ee)
    return out.reshape(M, N, D)


def kernel(batch_mask, mask_emb):
    return tc_kernel(batch_mask, mask_emb)
